# Initial kernel scaffold; baseline (speedup 1.0000x reference)
import jax
import jax.numpy as jnp
from jax.experimental import pallas as pl


def kernel(x, edge_index, edge_attr, W1, b1, W2, b2, Wf1, bf1, Wf2, bf2):
    raise NotImplementedError("write your pallas kernel here")


# speedup vs baseline: 20.7498x; 20.7498x over previous
import functools

import jax
import jax.numpy as jnp
from jax import lax
from jax.experimental import pallas as pl
from jax.experimental.pallas import tpu as pltpu
from jax.experimental.pallas import tpu_sc as plsc

_N = 10000
_NP = 10240
_E = 320000
_D = 32
_DIN = 128
_NCLS = 10
_NC, _NS = 2, 16
_NW = _NC * _NS
_CH = 128
_EP = 331776
_CPW = _EP // (_NW * _CH)
_RPW = _NP // _NS
_NPF = _NP * _D
_FPW = _NPF // _NW


def _worker(cid, sid):
    return sid * _NC + cid


def _lane_bcast(v, j):
    idx = jnp.full((16, 1), j, jnp.int32)
    dn = lax.GatherDimensionNumbers(offset_dims=(), collapsed_slice_dims=(0,),
                                    start_index_map=(0,))
    return lax.gather(v, idx, dn, (1,),
                      mode=lax.GatherScatterMode.PROMISE_IN_BOUNDS)


def _rsqrt16(d):
    i = plsc.bitcast(d, jnp.int32)
    i = jnp.int32(0x5F3759DF) - lax.shift_right_arithmetic(i, 1)
    y = plsc.bitcast(i, jnp.float32)
    for _ in range(3):
        y = y * (1.5 - 0.5 * d * y * y)
    return y


_NR = _NP // _CH
_RPS = _NR // _NS


def _deg_body(col_hbm, ew_hbm, degp_hbm, col_v, ew_v, accf, acc2d, idx_v,
              acc_sh):
    cid = lax.axis_index("c")
    sid = lax.axis_index("s")
    w = _worker(cid, sid)
    pltpu.sync_copy(col_hbm.at[w], col_v)
    pltpu.sync_copy(ew_hbm.at[w], ew_v)

    def zero(i, _):
        accf[pl.ds(i * 16, 16)] = jnp.zeros((16,), jnp.float32)
        return 0

    lax.fori_loop(0, _NP // 16, zero, 0)

    def zero2(r, _):
        for g in range(8):
            acc2d[r, pl.ds(g * 16, 16)] = jnp.zeros((16,), jnp.float32)
        return 0

    lax.fori_loop(0, _NR, zero2, 0)
    for k in range(_NR // 16):
        idx_v[pl.ds(k * 16, 16)] = lax.iota(jnp.int32, 16) + (k * 16)
    @pl.when(sid < _NR // 8)
    def _():
        pltpu.sync_copy(acc2d.at[pl.ds(sid * 8, 8), :],
                        acc_sh.at[pl.ds(sid * 8, 8), :])

    def body(ci, _):
        for g in range(8):
            idx = col_v[ci, pl.ds(g * 16, 16)]
            val = ew_v[ci, pl.ds(g * 16, 16)]
            plsc.addupdate_scatter(accf, [idx], val, mask=idx >= 0)
        return 0

    lax.fori_loop(0, _CPW, body, 0)

    def stage(r, _):
        for g in range(8):
            acc2d[r, pl.ds(g * 16, 16)] = accf[pl.ds(r * _CH + g * 16, 16)]
        return 0

    lax.fori_loop(0, _NR, stage, 0)
    plsc.subcore_barrier()
    pltpu.sync_copy(acc2d, acc_sh.at[idx_v], add=True)
    plsc.subcore_barrier()
    @pl.when(sid < _NR // 8)
    def _():
        pltpu.sync_copy(acc_sh.at[pl.ds(sid * 8, 8), :],
                        acc2d.at[pl.ds(0, 8), :])
        pltpu.sync_copy(acc2d.at[pl.ds(0, 8), :],
                        degp_hbm.at[cid, pl.ds(sid * 8, 8), :])


def _norm_body(degp_hbm, row_hbm, col_hbm, ew_hbm, norm_hbm,
               d0_v, d1_v, disf, row_v, col_v, ew_v, nrm_v):
    cid = lax.axis_index("c")
    sid = lax.axis_index("s")
    w = _worker(cid, sid)
    pltpu.sync_copy(degp_hbm.at[0], d0_v)
    pltpu.sync_copy(degp_hbm.at[1], d1_v)

    def disb(r, _):
        for g in range(8):
            s = pl.ds(g * 16, 16)
            disf[pl.ds(r * _CH + g * 16, 16)] = _rsqrt16(
                d0_v[r, s] + d1_v[r, s])
        return 0

    lax.fori_loop(0, _NR, disb, 0)
    pltpu.sync_copy(row_hbm.at[w], row_v)
    pltpu.sync_copy(col_hbm.at[w], col_v)
    pltpu.sync_copy(ew_hbm.at[w], ew_v)

    def body(ci, _):
        for g in range(8):
            s = pl.ds(g * 16, 16)
            r = row_v[ci, s]
            c = col_v[ci, s]
            dr = plsc.load_gather(disf, [r])
            dc = plsc.load_gather(disf, [c])
            nrm_v[ci, s] = dr * ew_v[ci, s] * dc
        return 0

    lax.fori_loop(0, _CPW, body, 0)
    pltpu.sync_copy(nrm_v, norm_hbm.at[w])


def _hop_body(h_hbm, row_hbm, col_hbm, nrm_hbm, part_hbm,
              row_v, col_v, nrm_v, gbuf, zbuf, acc_sh, sem):
    cid = lax.axis_index("c")
    sid = lax.axis_index("s")
    w = _worker(cid, sid)
    pltpu.sync_copy(row_hbm.at[w], row_v)
    pltpu.sync_copy(col_hbm.at[w], col_v)
    pltpu.sync_copy(nrm_hbm.at[w], nrm_v)

    def zero(i, _):
        zbuf[i, pl.ds(0, 16)] = jnp.zeros((16,), jnp.float32)
        zbuf[i, pl.ds(16, 16)] = jnp.zeros((16,), jnp.float32)
        return 0

    lax.fori_loop(0, _CH, zero, 0)
    for k in range(_RPW // _CH):
        pltpu.sync_copy(zbuf, acc_sh.at[pl.ds(sid * _RPW + k * _CH, _CH), :])
    plsc.subcore_barrier()

    def body(ci, _):
        pltpu.async_copy(h_hbm.at[row_v.at[ci]], gbuf, sem).wait()
        for g in range(8):
            nv = nrm_v[ci, pl.ds(g * 16, 16)]
            for j in range(16):
                e = g * 16 + j
                b = _lane_bcast(nv, j)
                gbuf[e, pl.ds(0, 16)] = gbuf[e, pl.ds(0, 16)] * b
                gbuf[e, pl.ds(16, 16)] = gbuf[e, pl.ds(16, 16)] * b
        pltpu.sync_copy(gbuf, acc_sh.at[col_v.at[ci]], add=True)
        return 0

    lax.fori_loop(0, _CPW, body, 0)
    plsc.subcore_barrier()
    for k in range(_RPW // _CH):
        r = sid * _RPW + k * _CH
        pltpu.sync_copy(acc_sh.at[pl.ds(r, _CH), :], gbuf)
        pltpu.sync_copy(gbuf, part_hbm.at[cid, pl.ds(r, _CH), :])


def _comb_body(with_elu, p_hbm, b_hbm, h_hbm, p0_v, p1_v, o_v, b_v):
    cid = lax.axis_index("c")
    sid = lax.axis_index("s")
    base = _worker(cid, sid) * _FPW
    pltpu.sync_copy(p_hbm.at[0, pl.ds(base, _FPW)], p0_v)
    pltpu.sync_copy(p_hbm.at[1, pl.ds(base, _FPW)], p1_v)
    if with_elu:
        pltpu.sync_copy(b_hbm, b_v)
        blo = b_v[pl.ds(0, 16)]
        bhi = b_v[pl.ds(16, 16)]

    def body(i, _):
        for sub in range(2):
            s = pl.ds(i * 32 + sub * 16, 16)
            v = p0_v[s] + p1_v[s]
            if with_elu:
                v = v + (blo if sub == 0 else bhi)
                v = jnp.where(v > 0, v, jnp.exp(v) - 1.0)
            o_v[s] = v
        return 0

    lax.fori_loop(0, _FPW // 32, body, 0)
    pltpu.sync_copy(o_v, h_hbm.at[pl.ds(base, _FPW)])


def _mm_in_body(x_ref, w_ref, o_ref):
    o_ref[...] = jnp.dot(x_ref[...], w_ref[...],
                         preferred_element_type=jnp.float32)


def _fc_body(p_ref, w2_ref, b2_ref, wf1_ref, bf1_ref, wf2_ref, bf2_ref, o_ref):
    h2 = jnp.dot(p_ref[...], w2_ref[...],
                 preferred_element_type=jnp.float32) + b2_ref[...]
    h2 = jnp.where(h2 > 0, h2, jnp.exp(h2) - 1.0)
    h3 = jnp.dot(h2, wf1_ref[...],
                 preferred_element_type=jnp.float32) + bf1_ref[...]
    h3 = jnp.where(h3 > 0, h3, jnp.exp(h3) - 1.0)
    lg = jnp.dot(h3, wf2_ref[...],
                 preferred_element_type=jnp.float32) + bf2_ref[...]
    valid = lax.broadcasted_iota(jnp.int32, lg.shape, 1) < _NCLS
    lgm = jnp.where(valid, lg, -jnp.inf)
    m = jnp.max(lgm, axis=1, keepdims=True)
    ex = jnp.where(valid, jnp.exp(lgm - m), 0.0)
    lse = jnp.log(jnp.sum(ex, axis=1, keepdims=True)) + m
    o_ref[...] = lgm - lse


def kernel(x, edge_index, edge_attr, W1, b1, W2, b2, Wf1, bf1, Wf2, bf2):
    f32 = jnp.float32
    row = edge_index[0].astype(jnp.int32)
    col = edge_index[1].astype(jnp.int32)
    ew = edge_attr[:, 0].astype(f32)

    pad = _EP - _E - _NP
    loops = jnp.arange(_NP, dtype=jnp.int32)
    zi = jnp.zeros((pad,), jnp.int32)
    esh = (_NW, _CPW, _CH)
    row2d = jnp.concatenate([row, loops, zi]).reshape(esh)
    col2d = jnp.concatenate([col, loops, zi]).reshape(esh)
    ew2d = jnp.concatenate(
        [ew, jnp.ones((_NP,), f32), jnp.zeros((pad,), f32)]
    ).reshape(esh)

    xp = jnp.zeros((_NP, _DIN), f32).at[:_N].set(x)

    mesh = plsc.VectorSubcoreMesh(core_axis_name="c", subcore_axis_name="s",
                                  num_cores=_NC, num_subcores=_NS)

    y = pl.pallas_call(
        _mm_in_body,
        grid=(_NP // 512,),
        in_specs=[pl.BlockSpec((512, _DIN), lambda i: (i, 0)),
                  pl.BlockSpec((_DIN, _D), lambda i: (0, 0))],
        out_specs=pl.BlockSpec((512, _D), lambda i: (i, 0)),
        out_shape=jax.ShapeDtypeStruct((_NP, _D), f32),
    )(xp, W1.T)

    degp = pl.kernel(
        _deg_body,
        out_type=jax.ShapeDtypeStruct((_NC, _NR, _CH), f32),
        mesh=mesh,
        compiler_params=pltpu.CompilerParams(needs_layout_passes=False),
        scratch_types=[
            pltpu.VMEM((_CPW, _CH), jnp.int32),
            pltpu.VMEM((_CPW, _CH), f32),
            pltpu.VMEM((_NP,), f32),
            pltpu.VMEM((_NR, _CH), f32),
            pltpu.VMEM((_NR,), jnp.int32),
            pltpu.VMEM_SHARED((_NR, _CH), f32),
        ],
    )(col2d, ew2d)

    norm2d = pl.kernel(
        _norm_body,
        out_type=jax.ShapeDtypeStruct(esh, f32),
        mesh=mesh,
        compiler_params=pltpu.CompilerParams(needs_layout_passes=False),
        scratch_types=[
            pltpu.VMEM((_NR, _CH), f32),
            pltpu.VMEM((_NR, _CH), f32),
            pltpu.VMEM((_NP,), f32),
            pltpu.VMEM((_CPW, _CH), jnp.int32),
            pltpu.VMEM((_CPW, _CH), jnp.int32),
            pltpu.VMEM((_CPW, _CH), f32),
            pltpu.VMEM((_CPW, _CH), f32),
        ],
    )(degp, row2d, col2d, ew2d)

    hop = pl.kernel(
        _hop_body,
        out_type=jax.ShapeDtypeStruct((_NC, _NP, _D), f32),
        mesh=mesh,
        compiler_params=pltpu.CompilerParams(needs_layout_passes=False,
                                             use_tc_tiling_on_sc=False),
        scratch_types=[
            pltpu.VMEM((_CPW, _CH), jnp.int32),
            pltpu.VMEM((_CPW, _CH), jnp.int32),
            pltpu.VMEM((_CPW, _CH), f32),
            pltpu.VMEM((_CH, _D), f32),
            pltpu.VMEM((_CH, _D), f32),
            pltpu.VMEM_SHARED((_NP, _D), f32),
            pltpu.SemaphoreType.DMA,
        ],
    )

    comb_scratch = [
        pltpu.VMEM((_FPW,), f32),
        pltpu.VMEM((_FPW,), f32),
        pltpu.VMEM((_FPW,), f32),
        pltpu.VMEM((_D,), f32),
    ]
    comb_plain = pl.kernel(
        functools.partial(_comb_body, False),
        out_type=jax.ShapeDtypeStruct((_NPF,), f32),
        mesh=mesh,
        compiler_params=pltpu.CompilerParams(needs_layout_passes=False),
        scratch_types=comb_scratch,
    )
    comb_elu = pl.kernel(
        functools.partial(_comb_body, True),
        out_type=jax.ShapeDtypeStruct((_NPF,), f32),
        mesh=mesh,
        compiler_params=pltpu.CompilerParams(needs_layout_passes=False),
        scratch_types=comb_scratch,
    )

    h = y
    for hop_i in range(6):
        part = hop(h, row2d, col2d, norm2d)
        pf = part.reshape(_NC, _NPF)
        if hop_i == 2:
            h = comb_elu(pf, b1).reshape(_NP, _D)
        else:
            h = comb_plain(pf, b1).reshape(_NP, _D)

    w2t = W2.T
    wf1t = Wf1.T
    wf2t = jnp.zeros((128, 128), f32).at[:, :_NCLS].set(Wf2.T)
    bf2p = jnp.zeros((1, 128), f32).at[0, :_NCLS].set(bf2)
    out_full = pl.pallas_call(
        _fc_body,
        grid=(_NP // 512,),
        in_specs=[
            pl.BlockSpec((512, _D), lambda i: (i, 0)),
            pl.BlockSpec((_D, 64), lambda i: (0, 0)),
            pl.BlockSpec((1, 64), lambda i: (0, 0)),
            pl.BlockSpec((64, _DIN), lambda i: (0, 0)),
            pl.BlockSpec((1, _DIN), lambda i: (0, 0)),
            pl.BlockSpec((_DIN, 128), lambda i: (0, 0)),
            pl.BlockSpec((1, 128), lambda i: (0, 0)),
        ],
        out_specs=pl.BlockSpec((512, 128), lambda i: (i, 0)),
        out_shape=jax.ShapeDtypeStruct((_NP, 128), f32),
    )(h, w2t, b2.reshape(1, 64), wf1t, bf1.reshape(1, _DIN), wf2t, bf2p)

    return out_full[:_N, :_NCLS]


# speedup vs baseline: 21.5642x; 1.0392x over previous
import functools

import jax
import jax.numpy as jnp
from jax import lax
from jax.experimental import pallas as pl
from jax.experimental.pallas import tpu as pltpu
from jax.experimental.pallas import tpu_sc as plsc

_N = 10000
_NP = 10240
_E = 320000
_D = 32
_DIN = 128
_NCLS = 10
_NC, _NS = 2, 16
_NW = _NC * _NS
_CH = 128
_EP = 335872
_CPW = _EP // (_NW * _CH)
_RPW = _NP // _NS
_NPF = _NP * _D
_FPW = _NPF // _NW


def _worker(cid, sid):
    return sid * _NC + cid


def _lane_bcast(v, j):
    idx = jnp.full((16, 1), j, jnp.int32)
    dn = lax.GatherDimensionNumbers(offset_dims=(), collapsed_slice_dims=(0,),
                                    start_index_map=(0,))
    return lax.gather(v, idx, dn, (1,),
                      mode=lax.GatherScatterMode.PROMISE_IN_BOUNDS)


def _rsqrt16(d):
    i = plsc.bitcast(d, jnp.int32)
    i = jnp.int32(0x5F3759DF) - lax.shift_right_arithmetic(i, 1)
    y = plsc.bitcast(i, jnp.float32)
    for _ in range(3):
        y = y * (1.5 - 0.5 * d * y * y)
    return y


_NR = _NP // _CH
_RPS = _NR // _NS


def _deg_body(col_hbm, ew_hbm, degp_hbm, col_v, ew_v, accf, acc2d, idx_v,
              acc_sh):
    cid = lax.axis_index("c")
    sid = lax.axis_index("s")
    w = _worker(cid, sid)
    pltpu.sync_copy(col_hbm.at[w], col_v)
    pltpu.sync_copy(ew_hbm.at[w], ew_v)

    def zero(i, _):
        accf[pl.ds(i * 16, 16)] = jnp.zeros((16,), jnp.float32)
        return 0

    lax.fori_loop(0, _NP // 16, zero, 0)

    def zero2(r, _):
        for g in range(8):
            acc2d[r, pl.ds(g * 16, 16)] = jnp.zeros((16,), jnp.float32)
        return 0

    lax.fori_loop(0, _NR, zero2, 0)
    for k in range(_NR // 16):
        idx_v[pl.ds(k * 16, 16)] = lax.iota(jnp.int32, 16) + (k * 16)
    @pl.when(sid < _NR // 8)
    def _():
        pltpu.sync_copy(acc2d.at[pl.ds(sid * 8, 8), :],
                        acc_sh.at[pl.ds(sid * 8, 8), :])

    def body(ci, _):
        for g in range(8):
            idx = col_v[ci, pl.ds(g * 16, 16)]
            val = ew_v[ci, pl.ds(g * 16, 16)]
            plsc.addupdate_scatter(accf, [idx], val, mask=idx >= 0)
        return 0

    lax.fori_loop(0, _CPW, body, 0)

    def stage(r, _):
        for g in range(8):
            acc2d[r, pl.ds(g * 16, 16)] = accf[pl.ds(r * _CH + g * 16, 16)]
        return 0

    lax.fori_loop(0, _NR, stage, 0)
    plsc.subcore_barrier()
    pltpu.sync_copy(acc2d, acc_sh.at[idx_v], add=True)
    plsc.subcore_barrier()
    @pl.when(sid < _NR // 8)
    def _():
        pltpu.sync_copy(acc_sh.at[pl.ds(sid * 8, 8), :],
                        acc2d.at[pl.ds(0, 8), :])
        pltpu.sync_copy(acc2d.at[pl.ds(0, 8), :],
                        degp_hbm.at[cid, pl.ds(sid * 8, 8), :])


def _norm_body(degp_hbm, row_hbm, col_hbm, ew_hbm, norm_hbm,
               d0_v, d1_v, disf, row_v, col_v, ew_v, nrm_v):
    cid = lax.axis_index("c")
    sid = lax.axis_index("s")
    w = _worker(cid, sid)
    pltpu.sync_copy(degp_hbm.at[0], d0_v)
    pltpu.sync_copy(degp_hbm.at[1], d1_v)

    def disb(r, _):
        for g in range(8):
            s = pl.ds(g * 16, 16)
            disf[pl.ds(r * _CH + g * 16, 16)] = _rsqrt16(
                d0_v[r, s] + d1_v[r, s])
        return 0

    lax.fori_loop(0, _NR, disb, 0)
    pltpu.sync_copy(row_hbm.at[w], row_v)
    pltpu.sync_copy(col_hbm.at[w], col_v)
    pltpu.sync_copy(ew_hbm.at[w], ew_v)

    def body(ci, _):
        for g in range(8):
            s = pl.ds(g * 16, 16)
            r = row_v[ci, s]
            c = col_v[ci, s]
            dr = plsc.load_gather(disf, [r])
            dc = plsc.load_gather(disf, [c])
            nrm_v[ci, s] = dr * ew_v[ci, s] * dc
        return 0

    lax.fori_loop(0, _CPW, body, 0)
    pltpu.sync_copy(nrm_v, norm_hbm.at[w])


def _hop_body(h_hbm, row_hbm, col_hbm, nrm_hbm, part_hbm,
              row_v, col_v, nrm_v, gb0, gb1, zbuf, acc_sh, sem0, sem1):
    cid = lax.axis_index("c")
    sid = lax.axis_index("s")
    w = _worker(cid, sid)
    pltpu.sync_copy(row_hbm.at[w], row_v)
    pltpu.sync_copy(col_hbm.at[w], col_v)
    pltpu.sync_copy(nrm_hbm.at[w], nrm_v)

    def zero(i, _):
        zbuf[i, pl.ds(0, 16)] = jnp.zeros((16,), jnp.float32)
        zbuf[i, pl.ds(16, 16)] = jnp.zeros((16,), jnp.float32)
        return 0

    lax.fori_loop(0, _CH, zero, 0)
    for k in range(_RPW // _CH):
        pltpu.sync_copy(zbuf, acc_sh.at[pl.ds(sid * _RPW + k * _CH, _CH), :])
    plsc.subcore_barrier()

    gbufs = (gb0, gb1)
    sems = (sem0, sem1)
    pltpu.async_copy(h_hbm.at[row_v.at[0]], gb0, sem0)
    pltpu.async_copy(h_hbm.at[row_v.at[1]], gb1, sem1)

    @pl.loop(0, _CPW, step=2)
    def body(ci):
        for b in range(2):
            cur = ci + b
            gbuf = gbufs[b]
            pltpu.make_async_copy(h_hbm.at[row_v.at[cur]], gbuf,
                                  sems[b]).wait()
            for g in range(8):
                nv = nrm_v[cur, pl.ds(g * 16, 16)]
                for j in range(16):
                    e = g * 16 + j
                    bc = _lane_bcast(nv, j)
                    gbuf[e, pl.ds(0, 16)] = gbuf[e, pl.ds(0, 16)] * bc
                    gbuf[e, pl.ds(16, 16)] = gbuf[e, pl.ds(16, 16)] * bc
            pltpu.sync_copy(gbuf, acc_sh.at[col_v.at[cur]], add=True)

            @pl.when(cur + 2 < _CPW)
            def _():
                pltpu.async_copy(h_hbm.at[row_v.at[cur + 2]], gbuf, sems[b])

    plsc.subcore_barrier()
    for k in range(_RPW // _CH):
        r = sid * _RPW + k * _CH
        pltpu.sync_copy(acc_sh.at[pl.ds(r, _CH), :], gb0)
        pltpu.sync_copy(gb0, part_hbm.at[cid, pl.ds(r, _CH), :])


def _comb_body(with_elu, p_hbm, b_hbm, h_hbm, p0_v, p1_v, o_v, b_v):
    cid = lax.axis_index("c")
    sid = lax.axis_index("s")
    base = _worker(cid, sid) * _FPW
    pltpu.sync_copy(p_hbm.at[0, pl.ds(base, _FPW)], p0_v)
    pltpu.sync_copy(p_hbm.at[1, pl.ds(base, _FPW)], p1_v)
    if with_elu:
        pltpu.sync_copy(b_hbm, b_v)
        blo = b_v[pl.ds(0, 16)]
        bhi = b_v[pl.ds(16, 16)]

    def body(i, _):
        for sub in range(2):
            s = pl.ds(i * 32 + sub * 16, 16)
            v = p0_v[s] + p1_v[s]
            if with_elu:
                v = v + (blo if sub == 0 else bhi)
                v = jnp.where(v > 0, v, jnp.exp(v) - 1.0)
            o_v[s] = v
        return 0

    lax.fori_loop(0, _FPW // 32, body, 0)
    pltpu.sync_copy(o_v, h_hbm.at[pl.ds(base, _FPW)])


def _mm_in_body(x_ref, w_ref, o_ref):
    o_ref[...] = jnp.dot(x_ref[...], w_ref[...],
                         preferred_element_type=jnp.float32)


def _fc_body(p_ref, w2_ref, b2_ref, wf1_ref, bf1_ref, wf2_ref, bf2_ref, o_ref):
    h2 = jnp.dot(p_ref[...], w2_ref[...],
                 preferred_element_type=jnp.float32) + b2_ref[...]
    h2 = jnp.where(h2 > 0, h2, jnp.exp(h2) - 1.0)
    h3 = jnp.dot(h2, wf1_ref[...],
                 preferred_element_type=jnp.float32) + bf1_ref[...]
    h3 = jnp.where(h3 > 0, h3, jnp.exp(h3) - 1.0)
    lg = jnp.dot(h3, wf2_ref[...],
                 preferred_element_type=jnp.float32) + bf2_ref[...]
    valid = lax.broadcasted_iota(jnp.int32, lg.shape, 1) < _NCLS
    lgm = jnp.where(valid, lg, -jnp.inf)
    m = jnp.max(lgm, axis=1, keepdims=True)
    ex = jnp.where(valid, jnp.exp(lgm - m), 0.0)
    lse = jnp.log(jnp.sum(ex, axis=1, keepdims=True)) + m
    o_ref[...] = lgm - lse


def kernel(x, edge_index, edge_attr, W1, b1, W2, b2, Wf1, bf1, Wf2, bf2):
    f32 = jnp.float32
    row = edge_index[0].astype(jnp.int32)
    col = edge_index[1].astype(jnp.int32)
    ew = edge_attr[:, 0].astype(f32)

    pad = _EP - _E - _NP
    loops = jnp.arange(_NP, dtype=jnp.int32)
    zi = jnp.zeros((pad,), jnp.int32)
    esh = (_NW, _CPW, _CH)
    row2d = jnp.concatenate([row, loops, zi]).reshape(esh)
    col2d = jnp.concatenate([col, loops, zi]).reshape(esh)
    ew2d = jnp.concatenate(
        [ew, jnp.ones((_NP,), f32), jnp.zeros((pad,), f32)]
    ).reshape(esh)

    xp = jnp.zeros((_NP, _DIN), f32).at[:_N].set(x)

    mesh = plsc.VectorSubcoreMesh(core_axis_name="c", subcore_axis_name="s",
                                  num_cores=_NC, num_subcores=_NS)

    y = pl.pallas_call(
        _mm_in_body,
        grid=(_NP // 512,),
        in_specs=[pl.BlockSpec((512, _DIN), lambda i: (i, 0)),
                  pl.BlockSpec((_DIN, _D), lambda i: (0, 0))],
        out_specs=pl.BlockSpec((512, _D), lambda i: (i, 0)),
        out_shape=jax.ShapeDtypeStruct((_NP, _D), f32),
    )(xp, W1.T)

    degp = pl.kernel(
        _deg_body,
        out_type=jax.ShapeDtypeStruct((_NC, _NR, _CH), f32),
        mesh=mesh,
        compiler_params=pltpu.CompilerParams(needs_layout_passes=False),
        scratch_types=[
            pltpu.VMEM((_CPW, _CH), jnp.int32),
            pltpu.VMEM((_CPW, _CH), f32),
            pltpu.VMEM((_NP,), f32),
            pltpu.VMEM((_NR, _CH), f32),
            pltpu.VMEM((_NR,), jnp.int32),
            pltpu.VMEM_SHARED((_NR, _CH), f32),
        ],
    )(col2d, ew2d)

    norm2d = pl.kernel(
        _norm_body,
        out_type=jax.ShapeDtypeStruct(esh, f32),
        mesh=mesh,
        compiler_params=pltpu.CompilerParams(needs_layout_passes=False),
        scratch_types=[
            pltpu.VMEM((_NR, _CH), f32),
            pltpu.VMEM((_NR, _CH), f32),
            pltpu.VMEM((_NP,), f32),
            pltpu.VMEM((_CPW, _CH), jnp.int32),
            pltpu.VMEM((_CPW, _CH), jnp.int32),
            pltpu.VMEM((_CPW, _CH), f32),
            pltpu.VMEM((_CPW, _CH), f32),
        ],
    )(degp, row2d, col2d, ew2d)

    hop = pl.kernel(
        _hop_body,
        out_type=jax.ShapeDtypeStruct((_NC, _NP, _D), f32),
        mesh=mesh,
        compiler_params=pltpu.CompilerParams(needs_layout_passes=False,
                                             use_tc_tiling_on_sc=False),
        scratch_types=[
            pltpu.VMEM((_CPW, _CH), jnp.int32),
            pltpu.VMEM((_CPW, _CH), jnp.int32),
            pltpu.VMEM((_CPW, _CH), f32),
            pltpu.VMEM((_CH, _D), f32),
            pltpu.VMEM((_CH, _D), f32),
            pltpu.VMEM((_CH, _D), f32),
            pltpu.VMEM_SHARED((_NP, _D), f32),
            pltpu.SemaphoreType.DMA,
            pltpu.SemaphoreType.DMA,
        ],
    )

    comb_scratch = [
        pltpu.VMEM((_FPW,), f32),
        pltpu.VMEM((_FPW,), f32),
        pltpu.VMEM((_FPW,), f32),
        pltpu.VMEM((_D,), f32),
    ]
    comb_plain = pl.kernel(
        functools.partial(_comb_body, False),
        out_type=jax.ShapeDtypeStruct((_NPF,), f32),
        mesh=mesh,
        compiler_params=pltpu.CompilerParams(needs_layout_passes=False),
        scratch_types=comb_scratch,
    )
    comb_elu = pl.kernel(
        functools.partial(_comb_body, True),
        out_type=jax.ShapeDtypeStruct((_NPF,), f32),
        mesh=mesh,
        compiler_params=pltpu.CompilerParams(needs_layout_passes=False),
        scratch_types=comb_scratch,
    )

    h = y
    for hop_i in range(6):
        part = hop(h, row2d, col2d, norm2d)
        pf = part.reshape(_NC, _NPF)
        if hop_i == 2:
            h = comb_elu(pf, b1).reshape(_NP, _D)
        else:
            h = comb_plain(pf, b1).reshape(_NP, _D)

    w2t = W2.T
    wf1t = Wf1.T
    wf2t = jnp.zeros((128, 128), f32).at[:, :_NCLS].set(Wf2.T)
    bf2p = jnp.zeros((1, 128), f32).at[0, :_NCLS].set(bf2)
    out_full = pl.pallas_call(
        _fc_body,
        grid=(_NP // 512,),
        in_specs=[
            pl.BlockSpec((512, _D), lambda i: (i, 0)),
            pl.BlockSpec((_D, 64), lambda i: (0, 0)),
            pl.BlockSpec((1, 64), lambda i: (0, 0)),
            pl.BlockSpec((64, _DIN), lambda i: (0, 0)),
            pl.BlockSpec((1, _DIN), lambda i: (0, 0)),
            pl.BlockSpec((_DIN, 128), lambda i: (0, 0)),
            pl.BlockSpec((1, 128), lambda i: (0, 0)),
        ],
        out_specs=pl.BlockSpec((512, 128), lambda i: (i, 0)),
        out_shape=jax.ShapeDtypeStruct((_NP, 128), f32),
    )(h, w2t, b2.reshape(1, 64), wf1t, bf1.reshape(1, _DIN), wf2t, bf2p)

    return out_full[:_N, :_NCLS]


# speedup vs baseline: 36.7141x; 1.7026x over previous
import functools

import jax
import jax.numpy as jnp
from jax import lax
from jax.experimental import pallas as pl
from jax.experimental.pallas import tpu as pltpu
from jax.experimental.pallas import tpu_sc as plsc

_N = 10000
_NP = 10240
_E = 320000
_D = 32
_DIN = 128
_NCLS = 10
_NC, _NS = 2, 16
_NW = _NC * _NS
_CH = 128
_EP = 335872
_CPW = _EP // (_NW * _CH)
_RPW = _NP // _NS
_NPF = _NP * _D
_FPW = _NPF // _NW


def _worker(cid, sid):
    return sid * _NC + cid


def _lane_bcast(v, j):
    idx = jnp.full((16, 1), j, jnp.int32)
    dn = lax.GatherDimensionNumbers(offset_dims=(), collapsed_slice_dims=(0,),
                                    start_index_map=(0,))
    return lax.gather(v, idx, dn, (1,),
                      mode=lax.GatherScatterMode.PROMISE_IN_BOUNDS)


def _rsqrt16(d):
    i = plsc.bitcast(d, jnp.int32)
    i = jnp.int32(0x5F3759DF) - lax.shift_right_arithmetic(i, 1)
    y = plsc.bitcast(i, jnp.float32)
    for _ in range(3):
        y = y * (1.5 - 0.5 * d * y * y)
    return y


_NR = _NP // _CH
_RPS = _NR // _NS


def _deg_body(col_hbm, ew_hbm, degp_hbm, col_v, ew_v, accf, acc2d, idx_v,
              acc_sh):
    cid = lax.axis_index("c")
    sid = lax.axis_index("s")
    w = _worker(cid, sid)
    pltpu.sync_copy(col_hbm.at[w], col_v)
    pltpu.sync_copy(ew_hbm.at[w], ew_v)

    def zero(i, _):
        accf[pl.ds(i * 16, 16)] = jnp.zeros((16,), jnp.float32)
        return 0

    lax.fori_loop(0, _NP // 16, zero, 0)

    def zero2(r, _):
        for g in range(8):
            acc2d[r, pl.ds(g * 16, 16)] = jnp.zeros((16,), jnp.float32)
        return 0

    lax.fori_loop(0, _NR, zero2, 0)
    for k in range(_NR // 16):
        idx_v[pl.ds(k * 16, 16)] = lax.iota(jnp.int32, 16) + (k * 16)
    @pl.when(sid < _NR // 8)
    def _():
        pltpu.sync_copy(acc2d.at[pl.ds(sid * 8, 8), :],
                        acc_sh.at[pl.ds(sid * 8, 8), :])

    def body(ci, _):
        for g in range(8):
            idx = col_v[ci, pl.ds(g * 16, 16)]
            val = ew_v[ci, pl.ds(g * 16, 16)]
            plsc.addupdate_scatter(accf, [idx], val, mask=idx >= 0)
        return 0

    lax.fori_loop(0, _CPW, body, 0)

    def stage(r, _):
        for g in range(8):
            acc2d[r, pl.ds(g * 16, 16)] = accf[pl.ds(r * _CH + g * 16, 16)]
        return 0

    lax.fori_loop(0, _NR, stage, 0)
    plsc.subcore_barrier()
    pltpu.sync_copy(acc2d, acc_sh.at[idx_v], add=True)
    plsc.subcore_barrier()
    @pl.when(sid < _NR // 8)
    def _():
        pltpu.sync_copy(acc_sh.at[pl.ds(sid * 8, 8), :],
                        acc2d.at[pl.ds(0, 8), :])
        pltpu.sync_copy(acc2d.at[pl.ds(0, 8), :],
                        degp_hbm.at[cid, pl.ds(sid * 8, 8), :])


def _norm_body(degp_hbm, row_hbm, col_hbm, ew_hbm, norm_hbm,
               d0_v, d1_v, disf, row_v, col_v, ew_v, nrm_v):
    cid = lax.axis_index("c")
    sid = lax.axis_index("s")
    w = _worker(cid, sid)
    pltpu.sync_copy(degp_hbm.at[0], d0_v)
    pltpu.sync_copy(degp_hbm.at[1], d1_v)

    def disb(r, _):
        for g in range(8):
            s = pl.ds(g * 16, 16)
            disf[pl.ds(r * _CH + g * 16, 16)] = _rsqrt16(
                d0_v[r, s] + d1_v[r, s])
        return 0

    lax.fori_loop(0, _NR, disb, 0)
    pltpu.sync_copy(row_hbm.at[w], row_v)
    pltpu.sync_copy(col_hbm.at[w], col_v)
    pltpu.sync_copy(ew_hbm.at[w], ew_v)

    def body(ci, _):
        for g in range(8):
            s = pl.ds(g * 16, 16)
            r = row_v[ci, s]
            c = col_v[ci, s]
            dr = plsc.load_gather(disf, [r])
            dc = plsc.load_gather(disf, [c])
            nrm_v[ci, s] = dr * ew_v[ci, s] * dc
        return 0

    lax.fori_loop(0, _CPW, body, 0)
    pltpu.sync_copy(nrm_v, norm_hbm.at[w])


def _hop_body(mode, *refs):
    if mode == "y":
        (h_hbm, row_hbm, col_hbm, nrm_hbm, part_hbm,
         row_v, col_v, nrm_v, gb0, gb1, zbuf, h_sh, acc_sh, sem0, sem1) = refs
    elif mode == "sum":
        (p_hbm, row_hbm, col_hbm, nrm_hbm, part_hbm,
         row_v, col_v, nrm_v, gb0, gb1, zbuf, h_sh, acc_sh, sb, ridx_v,
         sem0, sem1) = refs
    else:
        (p_hbm, b_hbm, row_hbm, col_hbm, nrm_hbm, part_hbm,
         row_v, col_v, nrm_v, gb0, gb1, zbuf, h_sh, acc_sh,
         eb0, eb1, b_v, sem0, sem1) = refs
    cid = lax.axis_index("c")
    sid = lax.axis_index("s")
    w = _worker(cid, sid)
    pltpu.sync_copy(row_hbm.at[w], row_v)
    pltpu.sync_copy(col_hbm.at[w], col_v)
    pltpu.sync_copy(nrm_hbm.at[w], nrm_v)

    rs = pl.ds(sid * _RPW, _RPW)
    if mode == "y":
        pltpu.sync_copy(h_hbm.at[rs, :], h_sh.at[rs, :])
    elif mode == "sum":
        pltpu.sync_copy(p_hbm.at[0, rs, :], h_sh.at[rs, :])
        pltpu.sync_copy(p_hbm.at[1, rs, :], sb)
        for k in range(_RPW // 16):
            ridx_v[pl.ds(k * 16, 16)] = (lax.iota(jnp.int32, 16)
                                         + (sid * _RPW + k * 16))
        pltpu.sync_copy(sb, h_sh.at[ridx_v], add=True)
    else:
        pltpu.sync_copy(p_hbm.at[0, rs, :], eb0)
        pltpu.sync_copy(p_hbm.at[1, rs, :], eb1)
        pltpu.sync_copy(b_hbm, b_v)
        blo = b_v[pl.ds(0, 16)]
        bhi = b_v[pl.ds(16, 16)]

        def elu_row(r, _):
            for sub in range(2):
                s = pl.ds(sub * 16, 16)
                v = eb0[r, s] + eb1[r, s] + (blo if sub == 0 else bhi)
                eb0[r, s] = jnp.where(v > 0, v, jnp.exp(v) - 1.0)
            return 0

        lax.fori_loop(0, _RPW, elu_row, 0)
        pltpu.sync_copy(eb0, h_sh.at[rs, :])

    def zero(i, _):
        zbuf[i, pl.ds(0, 16)] = jnp.zeros((16,), jnp.float32)
        zbuf[i, pl.ds(16, 16)] = jnp.zeros((16,), jnp.float32)
        return 0

    lax.fori_loop(0, _CH, zero, 0)
    for k in range(_RPW // _CH):
        pltpu.sync_copy(zbuf, acc_sh.at[pl.ds(sid * _RPW + k * _CH, _CH), :])
    plsc.subcore_barrier()

    gbufs = (gb0, gb1)
    sems = (sem0, sem1)
    pltpu.async_copy(h_sh.at[row_v.at[0]], gb0, sem0)
    pltpu.async_copy(h_sh.at[row_v.at[1]], gb1, sem1)

    @pl.loop(0, _CPW, step=2)
    def body(ci):
        for b in range(2):
            cur = ci + b
            gbuf = gbufs[b]
            pltpu.make_async_copy(h_sh.at[row_v.at[cur]], gbuf,
                                  sems[b]).wait()
            for g in range(8):
                nv = nrm_v[cur, pl.ds(g * 16, 16)]
                for j in range(16):
                    e = g * 16 + j
                    bc = _lane_bcast(nv, j)
                    gbuf[e, pl.ds(0, 16)] = gbuf[e, pl.ds(0, 16)] * bc
                    gbuf[e, pl.ds(16, 16)] = gbuf[e, pl.ds(16, 16)] * bc
            pltpu.sync_copy(gbuf, acc_sh.at[col_v.at[cur]], add=True)

            @pl.when(cur + 2 < _CPW)
            def _():
                pltpu.async_copy(h_sh.at[row_v.at[cur + 2]], gbuf, sems[b])

    plsc.subcore_barrier()
    for k in range(_RPW // _CH):
        r = sid * _RPW + k * _CH
        pltpu.sync_copy(acc_sh.at[pl.ds(r, _CH), :], gb0)
        pltpu.sync_copy(gb0, part_hbm.at[cid, pl.ds(r, _CH), :])


def _mm_in_body(x_ref, w_ref, o_ref):
    o_ref[...] = jnp.dot(x_ref[...], w_ref[...],
                         preferred_element_type=jnp.float32)


def _fc_body(p0_ref, p1_ref, w2_ref, b2_ref, wf1_ref, bf1_ref, wf2_ref,
             bf2_ref, o_ref):
    h2 = jnp.dot(p0_ref[...] + p1_ref[...], w2_ref[...],
                 preferred_element_type=jnp.float32) + b2_ref[...]
    h2 = jnp.where(h2 > 0, h2, jnp.exp(h2) - 1.0)
    h3 = jnp.dot(h2, wf1_ref[...],
                 preferred_element_type=jnp.float32) + bf1_ref[...]
    h3 = jnp.where(h3 > 0, h3, jnp.exp(h3) - 1.0)
    lg = jnp.dot(h3, wf2_ref[...],
                 preferred_element_type=jnp.float32) + bf2_ref[...]
    valid = lax.broadcasted_iota(jnp.int32, lg.shape, 1) < _NCLS
    lgm = jnp.where(valid, lg, -jnp.inf)
    m = jnp.max(lgm, axis=1, keepdims=True)
    ex = jnp.where(valid, jnp.exp(lgm - m), 0.0)
    lse = jnp.log(jnp.sum(ex, axis=1, keepdims=True)) + m
    o_ref[...] = lgm - lse


def kernel(x, edge_index, edge_attr, W1, b1, W2, b2, Wf1, bf1, Wf2, bf2):
    f32 = jnp.float32
    row = edge_index[0].astype(jnp.int32)
    col = edge_index[1].astype(jnp.int32)
    ew = edge_attr[:, 0].astype(f32)

    pad = _EP - _E - _NP
    loops = jnp.arange(_NP, dtype=jnp.int32)
    zi = jnp.zeros((pad,), jnp.int32)
    esh = (_NW, _CPW, _CH)
    row2d = jnp.concatenate([row, loops, zi]).reshape(esh)
    col2d = jnp.concatenate([col, loops, zi]).reshape(esh)
    ew2d = jnp.concatenate(
        [ew, jnp.ones((_NP,), f32), jnp.zeros((pad,), f32)]
    ).reshape(esh)

    xp = jnp.zeros((_NP, _DIN), f32).at[:_N].set(x)

    mesh = plsc.VectorSubcoreMesh(core_axis_name="c", subcore_axis_name="s",
                                  num_cores=_NC, num_subcores=_NS)

    y = pl.pallas_call(
        _mm_in_body,
        grid=(_NP // 512,),
        in_specs=[pl.BlockSpec((512, _DIN), lambda i: (i, 0)),
                  pl.BlockSpec((_DIN, _D), lambda i: (0, 0))],
        out_specs=pl.BlockSpec((512, _D), lambda i: (i, 0)),
        out_shape=jax.ShapeDtypeStruct((_NP, _D), f32),
    )(xp, W1.T)

    degp = pl.kernel(
        _deg_body,
        out_type=jax.ShapeDtypeStruct((_NC, _NR, _CH), f32),
        mesh=mesh,
        compiler_params=pltpu.CompilerParams(needs_layout_passes=False),
        scratch_types=[
            pltpu.VMEM((_CPW, _CH), jnp.int32),
            pltpu.VMEM((_CPW, _CH), f32),
            pltpu.VMEM((_NP,), f32),
            pltpu.VMEM((_NR, _CH), f32),
            pltpu.VMEM((_NR,), jnp.int32),
            pltpu.VMEM_SHARED((_NR, _CH), f32),
        ],
    )(col2d, ew2d)

    norm2d = pl.kernel(
        _norm_body,
        out_type=jax.ShapeDtypeStruct(esh, f32),
        mesh=mesh,
        compiler_params=pltpu.CompilerParams(needs_layout_passes=False),
        scratch_types=[
            pltpu.VMEM((_NR, _CH), f32),
            pltpu.VMEM((_NR, _CH), f32),
            pltpu.VMEM((_NP,), f32),
            pltpu.VMEM((_CPW, _CH), jnp.int32),
            pltpu.VMEM((_CPW, _CH), jnp.int32),
            pltpu.VMEM((_CPW, _CH), f32),
            pltpu.VMEM((_CPW, _CH), f32),
        ],
    )(degp, row2d, col2d, ew2d)

    hop_common_scratch = [
        pltpu.VMEM((_CPW, _CH), jnp.int32),
        pltpu.VMEM((_CPW, _CH), jnp.int32),
        pltpu.VMEM((_CPW, _CH), f32),
        pltpu.VMEM((_CH, _D), f32),
        pltpu.VMEM((_CH, _D), f32),
        pltpu.VMEM((_CH, _D), f32),
        pltpu.VMEM_SHARED((_NP, _D), f32),
        pltpu.VMEM_SHARED((_NP, _D), f32),
    ]
    sems = [pltpu.SemaphoreType.DMA, pltpu.SemaphoreType.DMA]
    hop_cp = pltpu.CompilerParams(needs_layout_passes=False,
                                  use_tc_tiling_on_sc=False)
    hop_out = jax.ShapeDtypeStruct((_NC, _NP, _D), f32)
    hop_y = pl.kernel(
        functools.partial(_hop_body, "y"),
        out_type=hop_out, mesh=mesh, compiler_params=hop_cp,
        scratch_types=hop_common_scratch + sems,
    )
    hop_sum = pl.kernel(
        functools.partial(_hop_body, "sum"),
        out_type=hop_out, mesh=mesh, compiler_params=hop_cp,
        scratch_types=hop_common_scratch + [
            pltpu.VMEM((_RPW, _D), f32),
            pltpu.VMEM((_RPW,), jnp.int32),
        ] + sems,
    )
    hop_elu = pl.kernel(
        functools.partial(_hop_body, "sum_elu"),
        out_type=hop_out, mesh=mesh, compiler_params=hop_cp,
        scratch_types=hop_common_scratch + [
            pltpu.VMEM((_RPW, _D), f32),
            pltpu.VMEM((_RPW, _D), f32),
            pltpu.VMEM((_D,), f32),
        ] + sems,
    )

    part = hop_y(y, row2d, col2d, norm2d)
    part = hop_sum(part, row2d, col2d, norm2d)
    part = hop_sum(part, row2d, col2d, norm2d)
    part = hop_elu(part, b1, row2d, col2d, norm2d)
    part = hop_sum(part, row2d, col2d, norm2d)
    part = hop_sum(part, row2d, col2d, norm2d)

    w2t = W2.T
    wf1t = Wf1.T
    wf2t = jnp.zeros((128, 128), f32).at[:, :_NCLS].set(Wf2.T)
    bf2p = jnp.zeros((1, 128), f32).at[0, :_NCLS].set(bf2)
    out_full = pl.pallas_call(
        _fc_body,
        grid=(_NP // 512,),
        in_specs=[
            pl.BlockSpec((512, _D), lambda i: (i, 0)),
            pl.BlockSpec((512, _D), lambda i: (i, 0)),
            pl.BlockSpec((_D, 64), lambda i: (0, 0)),
            pl.BlockSpec((1, 64), lambda i: (0, 0)),
            pl.BlockSpec((64, _DIN), lambda i: (0, 0)),
            pl.BlockSpec((1, _DIN), lambda i: (0, 0)),
            pl.BlockSpec((_DIN, 128), lambda i: (0, 0)),
            pl.BlockSpec((1, 128), lambda i: (0, 0)),
        ],
        out_specs=pl.BlockSpec((512, 128), lambda i: (i, 0)),
        out_shape=jax.ShapeDtypeStruct((_NP, 128), f32),
    )(part[0], part[1], w2t, b2.reshape(1, 64), wf1t, bf1.reshape(1, _DIN),
      wf2t, bf2p)

    return out_full[:_N, :_NCLS]


# speedup vs baseline: 38.5017x; 1.0487x over previous
import functools

import jax
import jax.numpy as jnp
from jax import lax
from jax.experimental import pallas as pl
from jax.experimental.pallas import tpu as pltpu
from jax.experimental.pallas import tpu_sc as plsc

_N = 10000
_NP = 10240
_E = 320000
_D = 32
_DIN = 128
_NCLS = 10
_NC, _NS = 2, 16
_NW = _NC * _NS
_CH = 128
_EP = 335872
_CPW = _EP // (_NW * _CH)
_RPW = _NP // _NS
_NPF = _NP * _D
_FPW = _NPF // _NW


def _worker(cid, sid):
    return sid * _NC + cid


def _lane_bcast(v, j):
    idx = jnp.full((16, 1), j, jnp.int32)
    dn = lax.GatherDimensionNumbers(offset_dims=(), collapsed_slice_dims=(0,),
                                    start_index_map=(0,))
    return lax.gather(v, idx, dn, (1,),
                      mode=lax.GatherScatterMode.PROMISE_IN_BOUNDS)


def _rsqrt16(d):
    i = plsc.bitcast(d, jnp.int32)
    i = jnp.int32(0x5F3759DF) - lax.shift_right_arithmetic(i, 1)
    y = plsc.bitcast(i, jnp.float32)
    for _ in range(3):
        y = y * (1.5 - 0.5 * d * y * y)
    return y


_NR = _NP // _CH
_RPS = _NR // _NS


def _deg_body(col_hbm, ew_hbm, degp_hbm, col_v, ew_v, accf, acc2d, idx_v,
              acc_sh):
    cid = lax.axis_index("c")
    sid = lax.axis_index("s")
    w = _worker(cid, sid)
    pltpu.sync_copy(col_hbm.at[w], col_v)
    pltpu.sync_copy(ew_hbm.at[w], ew_v)

    def zero(i, _):
        accf[pl.ds(i * 16, 16)] = jnp.zeros((16,), jnp.float32)
        return 0

    lax.fori_loop(0, _NP // 16, zero, 0)

    def zero2(r, _):
        for g in range(8):
            acc2d[r, pl.ds(g * 16, 16)] = jnp.zeros((16,), jnp.float32)
        return 0

    lax.fori_loop(0, _NR, zero2, 0)
    for k in range(_NR // 16):
        idx_v[pl.ds(k * 16, 16)] = lax.iota(jnp.int32, 16) + (k * 16)
    @pl.when(sid < _NR // 8)
    def _():
        pltpu.sync_copy(acc2d.at[pl.ds(sid * 8, 8), :],
                        acc_sh.at[pl.ds(sid * 8, 8), :])

    def body(ci, _):
        for g in range(8):
            idx = col_v[ci, pl.ds(g * 16, 16)]
            val = ew_v[ci, pl.ds(g * 16, 16)]
            plsc.addupdate_scatter(accf, [idx], val, mask=idx >= 0)
        return 0

    lax.fori_loop(0, _CPW, body, 0)

    def stage(r, _):
        for g in range(8):
            acc2d[r, pl.ds(g * 16, 16)] = accf[pl.ds(r * _CH + g * 16, 16)]
        return 0

    lax.fori_loop(0, _NR, stage, 0)
    plsc.subcore_barrier()
    pltpu.sync_copy(acc2d, acc_sh.at[idx_v], add=True)
    plsc.subcore_barrier()
    @pl.when(sid < _NR // 8)
    def _():
        pltpu.sync_copy(acc_sh.at[pl.ds(sid * 8, 8), :],
                        acc2d.at[pl.ds(0, 8), :])
        pltpu.sync_copy(acc2d.at[pl.ds(0, 8), :],
                        degp_hbm.at[cid, pl.ds(sid * 8, 8), :])


def _norm_body(degp_hbm, row_hbm, col_hbm, ew_hbm, norm_hbm,
               d0_v, d1_v, disf, row_v, col_v, ew_v, nrm_v):
    cid = lax.axis_index("c")
    sid = lax.axis_index("s")
    w = _worker(cid, sid)
    pltpu.sync_copy(degp_hbm.at[0], d0_v)
    pltpu.sync_copy(degp_hbm.at[1], d1_v)

    def disb(r, _):
        for g in range(8):
            s = pl.ds(g * 16, 16)
            disf[pl.ds(r * _CH + g * 16, 16)] = _rsqrt16(
                d0_v[r, s] + d1_v[r, s])
        return 0

    lax.fori_loop(0, _NR, disb, 0)
    pltpu.sync_copy(row_hbm.at[w], row_v)
    pltpu.sync_copy(col_hbm.at[w], col_v)
    pltpu.sync_copy(ew_hbm.at[w], ew_v)

    def body(ci, _):
        for g in range(8):
            s = pl.ds(g * 16, 16)
            r = row_v[ci, s]
            c = col_v[ci, s]
            dr = plsc.load_gather(disf, [r])
            dc = plsc.load_gather(disf, [c])
            nrm_v[ci, s] = dr * ew_v[ci, s] * dc
        return 0

    lax.fori_loop(0, _CPW, body, 0)
    pltpu.sync_copy(nrm_v, norm_hbm.at[w])


def _hop_body(mode, *refs):
    if mode == "y":
        (h_hbm, row_hbm, col_hbm, nrm_hbm, part_hbm,
         row_v, col_v, nrm_v, gb0, gb1, sb0, sb1, h_sh, acc_sh,
         gsem0, gsem1, ssem0, ssem1) = refs
    elif mode == "sum":
        (p_hbm, row_hbm, col_hbm, nrm_hbm, part_hbm,
         row_v, col_v, nrm_v, gb0, gb1, sb0, sb1, h_sh, acc_sh,
         ridx_v, gsem0, gsem1, ssem0, ssem1) = refs
    else:
        (p_hbm, b_hbm, row_hbm, col_hbm, nrm_hbm, part_hbm,
         row_v, col_v, nrm_v, gb0, gb1, sb0, sb1, h_sh, acc_sh,
         b_v, gsem0, gsem1, ssem0, ssem1) = refs
    cid = lax.axis_index("c")
    sid = lax.axis_index("s")
    w = _worker(cid, sid)
    pltpu.sync_copy(row_hbm.at[w], row_v)
    pltpu.sync_copy(col_hbm.at[w], col_v)
    pltpu.sync_copy(nrm_hbm.at[w], nrm_v)

    rs0 = sid * _RPW
    rs = pl.ds(rs0, _RPW)
    if mode == "y":
        pltpu.sync_copy(h_hbm.at[rs, :], h_sh.at[rs, :])
    elif mode == "sum":
        pltpu.sync_copy(p_hbm.at[0, rs, :], h_sh.at[rs, :])
        for k in range(_RPW // _CH):
            for g in range(_CH // 16):
                ridx_v[k, pl.ds(g * 16, 16)] = (lax.iota(jnp.int32, 16)
                                                + (rs0 + k * _CH + g * 16))
            pltpu.sync_copy(p_hbm.at[1, pl.ds(rs0 + k * _CH, _CH), :], gb0)
            pltpu.sync_copy(gb0, h_sh.at[ridx_v.at[k]], add=True)
    else:
        pltpu.sync_copy(b_hbm, b_v)
        blo = b_v[pl.ds(0, 16)]
        bhi = b_v[pl.ds(16, 16)]
        for k in range(_RPW // _CH):
            rk = pl.ds(rs0 + k * _CH, _CH)
            pltpu.sync_copy(p_hbm.at[0, rk, :], gb0)
            pltpu.sync_copy(p_hbm.at[1, rk, :], gb1)

            def elu_row(r, _):
                for sub in range(2):
                    s = pl.ds(sub * 16, 16)
                    v = gb0[r, s] + gb1[r, s] + (blo if sub == 0 else bhi)
                    sb0[r, s] = jnp.where(v > 0, v, jnp.exp(v) - 1.0)
                return 0

            lax.fori_loop(0, _CH, elu_row, 0)
            pltpu.sync_copy(sb0, h_sh.at[rk, :])

    def zero(i, _):
        sb1[i, pl.ds(0, 16)] = jnp.zeros((16,), jnp.float32)
        sb1[i, pl.ds(16, 16)] = jnp.zeros((16,), jnp.float32)
        return 0

    lax.fori_loop(0, _CH, zero, 0)
    for k in range(_RPW // _CH):
        pltpu.sync_copy(sb1, acc_sh.at[pl.ds(rs0 + k * _CH, _CH), :])
    plsc.subcore_barrier()

    gbufs = (gb0, gb1)
    sbufs = (sb0, sb1)
    gsems = (gsem0, gsem1)
    ssems = (ssem0, ssem1)
    pltpu.async_copy(h_sh.at[row_v.at[0]], gb0, gsem0)
    pltpu.async_copy(h_sh.at[row_v.at[1]], gb1, gsem1)

    @pl.loop(0, _CPW, step=2)
    def body(ci):
        for b in range(2):
            cur = ci + b
            gbuf = gbufs[b]
            sbuf = sbufs[b]
            pltpu.make_async_copy(h_sh.at[row_v.at[cur]], gbuf,
                                  gsems[b]).wait()

            @pl.when(cur >= 2)
            def _():
                pltpu.make_async_copy(sbuf, acc_sh.at[col_v.at[cur - 2]],
                                      ssems[b]).wait()

            for g in range(8):
                nv = nrm_v[cur, pl.ds(g * 16, 16)]
                for j in range(16):
                    e = g * 16 + j
                    bc = _lane_bcast(nv, j)
                    sbuf[e, pl.ds(0, 16)] = gbuf[e, pl.ds(0, 16)] * bc
                    sbuf[e, pl.ds(16, 16)] = gbuf[e, pl.ds(16, 16)] * bc
            pltpu.async_copy(sbuf, acc_sh.at[col_v.at[cur]], ssems[b],
                             add=True)

            @pl.when(cur + 2 < _CPW)
            def _():
                pltpu.async_copy(h_sh.at[row_v.at[cur + 2]], gbuf, gsems[b])

    for b in range(2):
        pltpu.make_async_copy(sbufs[b], acc_sh.at[col_v.at[_CPW - 2 + b]],
                              ssems[b]).wait()

    plsc.subcore_barrier()
    for k in range(_RPW // _CH):
        r = sid * _RPW + k * _CH
        pltpu.sync_copy(acc_sh.at[pl.ds(r, _CH), :], gb0)
        pltpu.sync_copy(gb0, part_hbm.at[cid, pl.ds(r, _CH), :])


def _mm_in_body(x_ref, w_ref, o_ref):
    o_ref[...] = jnp.dot(x_ref[...], w_ref[...],
                         preferred_element_type=jnp.float32)


def _fc_body(p0_ref, p1_ref, w2_ref, b2_ref, wf1_ref, bf1_ref, wf2_ref,
             bf2_ref, o_ref):
    h2 = jnp.dot(p0_ref[...] + p1_ref[...], w2_ref[...],
                 preferred_element_type=jnp.float32) + b2_ref[...]
    h2 = jnp.where(h2 > 0, h2, jnp.exp(h2) - 1.0)
    h3 = jnp.dot(h2, wf1_ref[...],
                 preferred_element_type=jnp.float32) + bf1_ref[...]
    h3 = jnp.where(h3 > 0, h3, jnp.exp(h3) - 1.0)
    lg = jnp.dot(h3, wf2_ref[...],
                 preferred_element_type=jnp.float32) + bf2_ref[...]
    valid = lax.broadcasted_iota(jnp.int32, lg.shape, 1) < _NCLS
    lgm = jnp.where(valid, lg, -jnp.inf)
    m = jnp.max(lgm, axis=1, keepdims=True)
    ex = jnp.where(valid, jnp.exp(lgm - m), 0.0)
    lse = jnp.log(jnp.sum(ex, axis=1, keepdims=True)) + m
    o_ref[...] = lgm - lse


def kernel(x, edge_index, edge_attr, W1, b1, W2, b2, Wf1, bf1, Wf2, bf2):
    f32 = jnp.float32
    row = edge_index[0].astype(jnp.int32)
    col = edge_index[1].astype(jnp.int32)
    ew = edge_attr[:, 0].astype(f32)

    pad = _EP - _E - _NP
    loops = jnp.arange(_NP, dtype=jnp.int32)
    zi = jnp.zeros((pad,), jnp.int32)
    esh = (_NW, _CPW, _CH)
    row2d = jnp.concatenate([row, loops, zi]).reshape(esh)
    col2d = jnp.concatenate([col, loops, zi]).reshape(esh)
    ew2d = jnp.concatenate(
        [ew, jnp.ones((_NP,), f32), jnp.zeros((pad,), f32)]
    ).reshape(esh)

    xp = jnp.zeros((_NP, _DIN), f32).at[:_N].set(x)

    mesh = plsc.VectorSubcoreMesh(core_axis_name="c", subcore_axis_name="s",
                                  num_cores=_NC, num_subcores=_NS)

    y = pl.pallas_call(
        _mm_in_body,
        grid=(_NP // 512,),
        in_specs=[pl.BlockSpec((512, _DIN), lambda i: (i, 0)),
                  pl.BlockSpec((_DIN, _D), lambda i: (0, 0))],
        out_specs=pl.BlockSpec((512, _D), lambda i: (i, 0)),
        out_shape=jax.ShapeDtypeStruct((_NP, _D), f32),
    )(xp, W1.T)

    degp = pl.kernel(
        _deg_body,
        out_type=jax.ShapeDtypeStruct((_NC, _NR, _CH), f32),
        mesh=mesh,
        compiler_params=pltpu.CompilerParams(needs_layout_passes=False),
        scratch_types=[
            pltpu.VMEM((_CPW, _CH), jnp.int32),
            pltpu.VMEM((_CPW, _CH), f32),
            pltpu.VMEM((_NP,), f32),
            pltpu.VMEM((_NR, _CH), f32),
            pltpu.VMEM((_NR,), jnp.int32),
            pltpu.VMEM_SHARED((_NR, _CH), f32),
        ],
    )(col2d, ew2d)

    norm2d = pl.kernel(
        _norm_body,
        out_type=jax.ShapeDtypeStruct(esh, f32),
        mesh=mesh,
        compiler_params=pltpu.CompilerParams(needs_layout_passes=False),
        scratch_types=[
            pltpu.VMEM((_NR, _CH), f32),
            pltpu.VMEM((_NR, _CH), f32),
            pltpu.VMEM((_NP,), f32),
            pltpu.VMEM((_CPW, _CH), jnp.int32),
            pltpu.VMEM((_CPW, _CH), jnp.int32),
            pltpu.VMEM((_CPW, _CH), f32),
            pltpu.VMEM((_CPW, _CH), f32),
        ],
    )(degp, row2d, col2d, ew2d)

    hop_common_scratch = [
        pltpu.VMEM((_CPW, _CH), jnp.int32),
        pltpu.VMEM((_CPW, _CH), jnp.int32),
        pltpu.VMEM((_CPW, _CH), f32),
        pltpu.VMEM((_CH, _D), f32),
        pltpu.VMEM((_CH, _D), f32),
        pltpu.VMEM((_CH, _D), f32),
        pltpu.VMEM((_CH, _D), f32),
        pltpu.VMEM_SHARED((_NP, _D), f32),
        pltpu.VMEM_SHARED((_NP, _D), f32),
    ]
    sems = [pltpu.SemaphoreType.DMA, pltpu.SemaphoreType.DMA,
            pltpu.SemaphoreType.DMA, pltpu.SemaphoreType.DMA]
    hop_cp = pltpu.CompilerParams(needs_layout_passes=False,
                                  use_tc_tiling_on_sc=False)
    hop_out = jax.ShapeDtypeStruct((_NC, _NP, _D), f32)
    hop_y = pl.kernel(
        functools.partial(_hop_body, "y"),
        out_type=hop_out, mesh=mesh, compiler_params=hop_cp,
        scratch_types=hop_common_scratch + sems,
    )
    hop_sum = pl.kernel(
        functools.partial(_hop_body, "sum"),
        out_type=hop_out, mesh=mesh, compiler_params=hop_cp,
        scratch_types=hop_common_scratch + [
            pltpu.VMEM((_RPW // _CH, _CH), jnp.int32),
        ] + sems,
    )
    hop_elu = pl.kernel(
        functools.partial(_hop_body, "sum_elu"),
        out_type=hop_out, mesh=mesh, compiler_params=hop_cp,
        scratch_types=hop_common_scratch + [
            pltpu.VMEM((_D,), f32),
        ] + sems,
    )

    part = hop_y(y, row2d, col2d, norm2d)
    part = hop_sum(part, row2d, col2d, norm2d)
    part = hop_sum(part, row2d, col2d, norm2d)
    part = hop_elu(part, b1, row2d, col2d, norm2d)
    part = hop_sum(part, row2d, col2d, norm2d)
    part = hop_sum(part, row2d, col2d, norm2d)

    w2t = W2.T
    wf1t = Wf1.T
    wf2t = jnp.zeros((128, 128), f32).at[:, :_NCLS].set(Wf2.T)
    bf2p = jnp.zeros((1, 128), f32).at[0, :_NCLS].set(bf2)
    out_full = pl.pallas_call(
        _fc_body,
        grid=(_NP // 512,),
        in_specs=[
            pl.BlockSpec((512, _D), lambda i: (i, 0)),
            pl.BlockSpec((512, _D), lambda i: (i, 0)),
            pl.BlockSpec((_D, 64), lambda i: (0, 0)),
            pl.BlockSpec((1, 64), lambda i: (0, 0)),
            pl.BlockSpec((64, _DIN), lambda i: (0, 0)),
            pl.BlockSpec((1, _DIN), lambda i: (0, 0)),
            pl.BlockSpec((_DIN, 128), lambda i: (0, 0)),
            pl.BlockSpec((1, 128), lambda i: (0, 0)),
        ],
        out_specs=pl.BlockSpec((512, 128), lambda i: (i, 0)),
        out_shape=jax.ShapeDtypeStruct((_NP, 128), f32),
    )(part[0], part[1], w2t, b2.reshape(1, 64), wf1t, bf1.reshape(1, _DIN),
      wf2t, bf2p)

    return out_full[:_N, :_NCLS]


# speedup vs baseline: 39.2167x; 1.0186x over previous
import functools

import jax
import jax.numpy as jnp
from jax import lax
from jax.experimental import pallas as pl
from jax.experimental.pallas import tpu as pltpu
from jax.experimental.pallas import tpu_sc as plsc

_N = 10000
_NP = 10240
_E = 320000
_D = 32
_DIN = 128
_NCLS = 10
_NC, _NS = 2, 16
_NW = _NC * _NS
_CH = 128
_EP = 335872
_CPW = _EP // (_NW * _CH)
_RPW = _NP // _NS
_NPF = _NP * _D
_FPW = _NPF // _NW


def _worker(cid, sid):
    return sid * _NC + cid


def _lane_bcast(v, j):
    idx = jnp.full((16, 1), j, jnp.int32)
    dn = lax.GatherDimensionNumbers(offset_dims=(), collapsed_slice_dims=(0,),
                                    start_index_map=(0,))
    return lax.gather(v, idx, dn, (1,),
                      mode=lax.GatherScatterMode.PROMISE_IN_BOUNDS)


def _rsqrt16(d):
    i = plsc.bitcast(d, jnp.int32)
    i = jnp.int32(0x5F3759DF) - lax.shift_right_arithmetic(i, 1)
    y = plsc.bitcast(i, jnp.float32)
    for _ in range(3):
        y = y * (1.5 - 0.5 * d * y * y)
    return y


_NR = _NP // _CH
_RPS = _NR // _NS


def _deg_body(col_hbm, ew_hbm, degp_hbm, col_v, ew_v, accf, acc2d, idx_v,
              acc_sh):
    cid = lax.axis_index("c")
    sid = lax.axis_index("s")
    w = _worker(cid, sid)
    pltpu.sync_copy(col_hbm.at[w], col_v)
    pltpu.sync_copy(ew_hbm.at[w], ew_v)

    def zero(i, _):
        accf[pl.ds(i * 16, 16)] = jnp.zeros((16,), jnp.float32)
        return 0

    lax.fori_loop(0, _NP // 16, zero, 0)

    def zero2(r, _):
        for g in range(8):
            acc2d[r, pl.ds(g * 16, 16)] = jnp.zeros((16,), jnp.float32)
        return 0

    lax.fori_loop(0, _NR, zero2, 0)
    for k in range(_NR // 16):
        idx_v[pl.ds(k * 16, 16)] = lax.iota(jnp.int32, 16) + (k * 16)
    @pl.when(sid < _NR // 8)
    def _():
        pltpu.sync_copy(acc2d.at[pl.ds(sid * 8, 8), :],
                        acc_sh.at[pl.ds(sid * 8, 8), :])

    def body(ci, _):
        for g in range(8):
            idx = col_v[ci, pl.ds(g * 16, 16)]
            val = ew_v[ci, pl.ds(g * 16, 16)]
            plsc.addupdate_scatter(accf, [idx], val, mask=idx >= 0)
        return 0

    lax.fori_loop(0, _CPW, body, 0)

    def stage(r, _):
        for g in range(8):
            acc2d[r, pl.ds(g * 16, 16)] = accf[pl.ds(r * _CH + g * 16, 16)]
        return 0

    lax.fori_loop(0, _NR, stage, 0)
    plsc.subcore_barrier()
    pltpu.sync_copy(acc2d, acc_sh.at[idx_v], add=True)
    plsc.subcore_barrier()
    @pl.when(sid < _NR // 8)
    def _():
        pltpu.sync_copy(acc_sh.at[pl.ds(sid * 8, 8), :],
                        acc2d.at[pl.ds(0, 8), :])
        pltpu.sync_copy(acc2d.at[pl.ds(0, 8), :],
                        degp_hbm.at[cid, pl.ds(sid * 8, 8), :])


def _norm_hop_body(degp_hbm, y_hbm, row_hbm, col_hbm, ew_hbm,
                   norm_hbm, part_hbm,
                   d0_v, d1_v, disf, row_v, col_v, nrm_v,
                   gb0, gb1, sb0, sb1, h_sh, acc_sh,
                   gsem0, gsem1, ssem0, ssem1):
    cid = lax.axis_index("c")
    sid = lax.axis_index("s")
    w = _worker(cid, sid)
    pltpu.sync_copy(degp_hbm.at[0], d0_v)
    pltpu.sync_copy(degp_hbm.at[1], d1_v)

    def disb(r, _):
        for g in range(8):
            s = pl.ds(g * 16, 16)
            disf[pl.ds(r * _CH + g * 16, 16)] = _rsqrt16(
                d0_v[r, s] + d1_v[r, s])
        return 0

    lax.fori_loop(0, _NR, disb, 0)
    pltpu.sync_copy(row_hbm.at[w], row_v)
    pltpu.sync_copy(col_hbm.at[w], col_v)
    pltpu.sync_copy(ew_hbm.at[w], nrm_v)

    def body(ci, _):
        for g in range(8):
            s = pl.ds(g * 16, 16)
            r = row_v[ci, s]
            c = col_v[ci, s]
            dr = plsc.load_gather(disf, [r])
            dc = plsc.load_gather(disf, [c])
            nrm_v[ci, s] = dr * nrm_v[ci, s] * dc
        return 0

    lax.fori_loop(0, _CPW, body, 0)
    pltpu.sync_copy(nrm_v, norm_hbm.at[w])

    rs0 = sid * _RPW
    rs = pl.ds(rs0, _RPW)
    pltpu.sync_copy(y_hbm.at[rs, :], h_sh.at[rs, :])

    def zero(i, _):
        sb1[i, pl.ds(0, 16)] = jnp.zeros((16,), jnp.float32)
        sb1[i, pl.ds(16, 16)] = jnp.zeros((16,), jnp.float32)
        return 0

    lax.fori_loop(0, _CH, zero, 0)
    for k in range(_RPW // _CH):
        pltpu.sync_copy(sb1, acc_sh.at[pl.ds(rs0 + k * _CH, _CH), :])
    plsc.subcore_barrier()

    gbufs = (gb0, gb1)
    sbufs = (sb0, sb1)
    gsems = (gsem0, gsem1)
    ssems = (ssem0, ssem1)
    pltpu.async_copy(h_sh.at[row_v.at[0]], gb0, gsem0)
    pltpu.async_copy(h_sh.at[row_v.at[1]], gb1, gsem1)

    @pl.loop(0, _CPW, step=2)
    def hbody(ci):
        for b in range(2):
            cur = ci + b
            gbuf = gbufs[b]
            sbuf = sbufs[b]
            pltpu.make_async_copy(h_sh.at[row_v.at[cur]], gbuf,
                                  gsems[b]).wait()

            @pl.when(cur >= 2)
            def _():
                pltpu.make_async_copy(sbuf, acc_sh.at[col_v.at[cur - 2]],
                                      ssems[b]).wait()

            for g in range(8):
                nv = nrm_v[cur, pl.ds(g * 16, 16)]
                for j in range(16):
                    e = g * 16 + j
                    bc = _lane_bcast(nv, j)
                    sbuf[e, pl.ds(0, 16)] = gbuf[e, pl.ds(0, 16)] * bc
                    sbuf[e, pl.ds(16, 16)] = gbuf[e, pl.ds(16, 16)] * bc
            pltpu.async_copy(sbuf, acc_sh.at[col_v.at[cur]], ssems[b],
                             add=True)

            @pl.when(cur + 2 < _CPW)
            def _():
                pltpu.async_copy(h_sh.at[row_v.at[cur + 2]], gbuf, gsems[b])

    for b in range(2):
        pltpu.make_async_copy(sbufs[b], acc_sh.at[col_v.at[_CPW - 2 + b]],
                              ssems[b]).wait()
    plsc.subcore_barrier()
    for k in range(_RPW // _CH):
        r = rs0 + k * _CH
        pltpu.sync_copy(acc_sh.at[pl.ds(r, _CH), :], gb0)
        pltpu.sync_copy(gb0, part_hbm.at[cid, pl.ds(r, _CH), :])


def _hop_body(mode, *refs):
    if mode == "y":
        (h_hbm, row_hbm, col_hbm, nrm_hbm, part_hbm,
         row_v, col_v, nrm_v, gb0, gb1, sb0, sb1, h_sh, acc_sh,
         gsem0, gsem1, ssem0, ssem1) = refs
    elif mode == "sum":
        (p_hbm, row_hbm, col_hbm, nrm_hbm, part_hbm,
         row_v, col_v, nrm_v, gb0, gb1, sb0, sb1, h_sh, acc_sh,
         ridx_v, gsem0, gsem1, ssem0, ssem1) = refs
    else:
        (p_hbm, b_hbm, row_hbm, col_hbm, nrm_hbm, part_hbm,
         row_v, col_v, nrm_v, gb0, gb1, sb0, sb1, h_sh, acc_sh,
         b_v, gsem0, gsem1, ssem0, ssem1) = refs
    cid = lax.axis_index("c")
    sid = lax.axis_index("s")
    w = _worker(cid, sid)
    pltpu.sync_copy(row_hbm.at[w], row_v)
    pltpu.sync_copy(col_hbm.at[w], col_v)
    pltpu.sync_copy(nrm_hbm.at[w], nrm_v)

    rs0 = sid * _RPW
    rs = pl.ds(rs0, _RPW)
    if mode == "y":
        pltpu.sync_copy(h_hbm.at[rs, :], h_sh.at[rs, :])
    elif mode == "sum":
        pltpu.sync_copy(p_hbm.at[0, rs, :], h_sh.at[rs, :])
        for k in range(_RPW // _CH):
            for g in range(_CH // 16):
                ridx_v[k, pl.ds(g * 16, 16)] = (lax.iota(jnp.int32, 16)
                                                + (rs0 + k * _CH + g * 16))
            pltpu.sync_copy(p_hbm.at[1, pl.ds(rs0 + k * _CH, _CH), :], gb0)
            pltpu.sync_copy(gb0, h_sh.at[ridx_v.at[k]], add=True)
    else:
        pltpu.sync_copy(b_hbm, b_v)
        blo = b_v[pl.ds(0, 16)]
        bhi = b_v[pl.ds(16, 16)]
        for k in range(_RPW // _CH):
            rk = pl.ds(rs0 + k * _CH, _CH)
            pltpu.sync_copy(p_hbm.at[0, rk, :], gb0)
            pltpu.sync_copy(p_hbm.at[1, rk, :], gb1)

            def elu_row(r, _):
                for sub in range(2):
                    s = pl.ds(sub * 16, 16)
                    v = gb0[r, s] + gb1[r, s] + (blo if sub == 0 else bhi)
                    sb0[r, s] = jnp.where(v > 0, v, jnp.exp(v) - 1.0)
                return 0

            lax.fori_loop(0, _CH, elu_row, 0)
            pltpu.sync_copy(sb0, h_sh.at[rk, :])

    def zero(i, _):
        sb1[i, pl.ds(0, 16)] = jnp.zeros((16,), jnp.float32)
        sb1[i, pl.ds(16, 16)] = jnp.zeros((16,), jnp.float32)
        return 0

    lax.fori_loop(0, _CH, zero, 0)
    for k in range(_RPW // _CH):
        pltpu.sync_copy(sb1, acc_sh.at[pl.ds(rs0 + k * _CH, _CH), :])
    plsc.subcore_barrier()

    gbufs = (gb0, gb1)
    sbufs = (sb0, sb1)
    gsems = (gsem0, gsem1)
    ssems = (ssem0, ssem1)
    pltpu.async_copy(h_sh.at[row_v.at[0]], gb0, gsem0)
    pltpu.async_copy(h_sh.at[row_v.at[1]], gb1, gsem1)

    @pl.loop(0, _CPW, step=2)
    def body(ci):
        for b in range(2):
            cur = ci + b
            gbuf = gbufs[b]
            sbuf = sbufs[b]
            pltpu.make_async_copy(h_sh.at[row_v.at[cur]], gbuf,
                                  gsems[b]).wait()

            @pl.when(cur >= 2)
            def _():
                pltpu.make_async_copy(sbuf, acc_sh.at[col_v.at[cur - 2]],
                                      ssems[b]).wait()

            for g in range(8):
                nv = nrm_v[cur, pl.ds(g * 16, 16)]
                for j in range(16):
                    e = g * 16 + j
                    bc = _lane_bcast(nv, j)
                    sbuf[e, pl.ds(0, 16)] = gbuf[e, pl.ds(0, 16)] * bc
                    sbuf[e, pl.ds(16, 16)] = gbuf[e, pl.ds(16, 16)] * bc
            pltpu.async_copy(sbuf, acc_sh.at[col_v.at[cur]], ssems[b],
                             add=True)

            @pl.when(cur + 2 < _CPW)
            def _():
                pltpu.async_copy(h_sh.at[row_v.at[cur + 2]], gbuf, gsems[b])

    for b in range(2):
        pltpu.make_async_copy(sbufs[b], acc_sh.at[col_v.at[_CPW - 2 + b]],
                              ssems[b]).wait()

    plsc.subcore_barrier()
    for k in range(_RPW // _CH):
        r = sid * _RPW + k * _CH
        pltpu.sync_copy(acc_sh.at[pl.ds(r, _CH), :], gb0)
        pltpu.sync_copy(gb0, part_hbm.at[cid, pl.ds(r, _CH), :])


def _mm_in_body(x_ref, w_ref, o_ref):
    o_ref[...] = jnp.dot(x_ref[...], w_ref[...],
                         preferred_element_type=jnp.float32)


def _fc_body(p0_ref, p1_ref, w2_ref, b2_ref, wf1_ref, bf1_ref, wf2_ref,
             bf2_ref, o_ref):
    h2 = jnp.dot(p0_ref[...] + p1_ref[...], w2_ref[...],
                 preferred_element_type=jnp.float32) + b2_ref[...]
    h2 = jnp.where(h2 > 0, h2, jnp.exp(h2) - 1.0)
    h3 = jnp.dot(h2, wf1_ref[...],
                 preferred_element_type=jnp.float32) + bf1_ref[...]
    h3 = jnp.where(h3 > 0, h3, jnp.exp(h3) - 1.0)
    lg = jnp.dot(h3, wf2_ref[...],
                 preferred_element_type=jnp.float32) + bf2_ref[...]
    valid = lax.broadcasted_iota(jnp.int32, lg.shape, 1) < _NCLS
    lgm = jnp.where(valid, lg, -jnp.inf)
    m = jnp.max(lgm, axis=1, keepdims=True)
    ex = jnp.where(valid, jnp.exp(lgm - m), 0.0)
    lse = jnp.log(jnp.sum(ex, axis=1, keepdims=True)) + m
    o_ref[...] = lgm - lse


def kernel(x, edge_index, edge_attr, W1, b1, W2, b2, Wf1, bf1, Wf2, bf2):
    f32 = jnp.float32
    row = edge_index[0].astype(jnp.int32)
    col = edge_index[1].astype(jnp.int32)
    ew = edge_attr[:, 0].astype(f32)

    pad = _EP - _E - _NP
    loops = jnp.arange(_NP, dtype=jnp.int32)
    zi = jnp.zeros((pad,), jnp.int32)
    esh = (_NW, _CPW, _CH)
    row2d = jnp.concatenate([row, loops, zi]).reshape(esh)
    col2d = jnp.concatenate([col, loops, zi]).reshape(esh)
    ew2d = jnp.concatenate(
        [ew, jnp.ones((_NP,), f32), jnp.zeros((pad,), f32)]
    ).reshape(esh)

    xp = jnp.zeros((_NP, _DIN), f32).at[:_N].set(x)

    mesh = plsc.VectorSubcoreMesh(core_axis_name="c", subcore_axis_name="s",
                                  num_cores=_NC, num_subcores=_NS)

    y = pl.pallas_call(
        _mm_in_body,
        grid=(_NP // 512,),
        in_specs=[pl.BlockSpec((512, _DIN), lambda i: (i, 0)),
                  pl.BlockSpec((_DIN, _D), lambda i: (0, 0))],
        out_specs=pl.BlockSpec((512, _D), lambda i: (i, 0)),
        out_shape=jax.ShapeDtypeStruct((_NP, _D), f32),
    )(xp, W1.T)

    degp = pl.kernel(
        _deg_body,
        out_type=jax.ShapeDtypeStruct((_NC, _NR, _CH), f32),
        mesh=mesh,
        compiler_params=pltpu.CompilerParams(needs_layout_passes=False),
        scratch_types=[
            pltpu.VMEM((_CPW, _CH), jnp.int32),
            pltpu.VMEM((_CPW, _CH), f32),
            pltpu.VMEM((_NP,), f32),
            pltpu.VMEM((_NR, _CH), f32),
            pltpu.VMEM((_NR,), jnp.int32),
            pltpu.VMEM_SHARED((_NR, _CH), f32),
        ],
    )(col2d, ew2d)

    norm2d, part = pl.kernel(
        _norm_hop_body,
        out_type=[jax.ShapeDtypeStruct(esh, f32),
                  jax.ShapeDtypeStruct((_NC, _NP, _D), f32)],
        mesh=mesh,
        compiler_params=pltpu.CompilerParams(needs_layout_passes=False,
                                             use_tc_tiling_on_sc=False),
        scratch_types=[
            pltpu.VMEM((_NR, _CH), f32),
            pltpu.VMEM((_NR, _CH), f32),
            pltpu.VMEM((_NP,), f32),
            pltpu.VMEM((_CPW, _CH), jnp.int32),
            pltpu.VMEM((_CPW, _CH), jnp.int32),
            pltpu.VMEM((_CPW, _CH), f32),
            pltpu.VMEM((_CH, _D), f32),
            pltpu.VMEM((_CH, _D), f32),
            pltpu.VMEM((_CH, _D), f32),
            pltpu.VMEM((_CH, _D), f32),
            pltpu.VMEM_SHARED((_NP, _D), f32),
            pltpu.VMEM_SHARED((_NP, _D), f32),
            pltpu.SemaphoreType.DMA, pltpu.SemaphoreType.DMA,
            pltpu.SemaphoreType.DMA, pltpu.SemaphoreType.DMA,
        ],
    )(degp, y, row2d, col2d, ew2d)

    hop_common_scratch = [
        pltpu.VMEM((_CPW, _CH), jnp.int32),
        pltpu.VMEM((_CPW, _CH), jnp.int32),
        pltpu.VMEM((_CPW, _CH), f32),
        pltpu.VMEM((_CH, _D), f32),
        pltpu.VMEM((_CH, _D), f32),
        pltpu.VMEM((_CH, _D), f32),
        pltpu.VMEM((_CH, _D), f32),
        pltpu.VMEM_SHARED((_NP, _D), f32),
        pltpu.VMEM_SHARED((_NP, _D), f32),
    ]
    sems = [pltpu.SemaphoreType.DMA, pltpu.SemaphoreType.DMA,
            pltpu.SemaphoreType.DMA, pltpu.SemaphoreType.DMA]
    hop_cp = pltpu.CompilerParams(needs_layout_passes=False,
                                  use_tc_tiling_on_sc=False)
    hop_out = jax.ShapeDtypeStruct((_NC, _NP, _D), f32)
    hop_sum = pl.kernel(
        functools.partial(_hop_body, "sum"),
        out_type=hop_out, mesh=mesh, compiler_params=hop_cp,
        scratch_types=hop_common_scratch + [
            pltpu.VMEM((_RPW // _CH, _CH), jnp.int32),
        ] + sems,
    )
    hop_elu = pl.kernel(
        functools.partial(_hop_body, "sum_elu"),
        out_type=hop_out, mesh=mesh, compiler_params=hop_cp,
        scratch_types=hop_common_scratch + [
            pltpu.VMEM((_D,), f32),
        ] + sems,
    )

    part = hop_sum(part, row2d, col2d, norm2d)
    part = hop_sum(part, row2d, col2d, norm2d)
    part = hop_elu(part, b1, row2d, col2d, norm2d)
    part = hop_sum(part, row2d, col2d, norm2d)
    part = hop_sum(part, row2d, col2d, norm2d)

    w2t = W2.T
    wf1t = Wf1.T
    wf2t = jnp.zeros((128, 128), f32).at[:, :_NCLS].set(Wf2.T)
    bf2p = jnp.zeros((1, 128), f32).at[0, :_NCLS].set(bf2)
    out_full = pl.pallas_call(
        _fc_body,
        grid=(_NP // 512,),
        in_specs=[
            pl.BlockSpec((512, _D), lambda i: (i, 0)),
            pl.BlockSpec((512, _D), lambda i: (i, 0)),
            pl.BlockSpec((_D, 64), lambda i: (0, 0)),
            pl.BlockSpec((1, 64), lambda i: (0, 0)),
            pl.BlockSpec((64, _DIN), lambda i: (0, 0)),
            pl.BlockSpec((1, _DIN), lambda i: (0, 0)),
            pl.BlockSpec((_DIN, 128), lambda i: (0, 0)),
            pl.BlockSpec((1, 128), lambda i: (0, 0)),
        ],
        out_specs=pl.BlockSpec((512, 128), lambda i: (i, 0)),
        out_shape=jax.ShapeDtypeStruct((_NP, 128), f32),
    )(part[0], part[1], w2t, b2.reshape(1, 64), wf1t, bf1.reshape(1, _DIN),
      wf2t, bf2p)

    return out_full[:_N, :_NCLS]
